# baseline (speedup 1.0000x reference)
import functools
import numpy as np

import jax
import jax.numpy as jnp
from jax import lax
from jax.experimental import pallas as pl
from jax.experimental.pallas import tpu as pltpu
from jax.experimental.pallas import tpu_sc as plsc

_N = 10000
_NACC = 10240
_E = 320000
_NC, _NS = 2, 16
_NW = _NC * _NS
_B = 128
_NB = 160
_EPAD = _NS * _NB * _B
_RPT = _NACC // _NS
_F = 128

_LAYERS = [(4, 24, 5, 2, 32), (24, 32, 5, 1, 32), (32, 64, 5, 1, 30), (64, 64, 3, 0, 28), (64, 96, 3, 0, 26)]
_OUT_LENS = [32, 30, 28, 26, 24]


def _pad128(n):
    return (n + 127) // 128 * 128


def _deg_body(rowp_hbm, deg_hbm, rowv, val, acc_sh, sem):
    ci = lax.axis_index("c")
    si = lax.axis_index("s")

    cp = pltpu.async_copy(rowp_hbm.at[si], rowv, sem)

    @pl.loop(0, _B)
    def _fill0(r):
        for j in range(_F // 16):
            val[r, pl.ds(16 * j, 16)] = jnp.zeros((16,), jnp.float32)

    @pl.loop(0, _RPT // _B)
    def _zero(z):
        pltpu.sync_copy(val, acc_sh.at[pl.ds(si * _RPT + z * _B, _B)])

    @pl.loop(0, _B)
    def _fill1(r):
        for j in range(_F // 16):
            val[r, pl.ds(16 * j, 16)] = jnp.ones((16,), jnp.float32)
    cp.wait()
    plsc.subcore_barrier()

    @pl.loop(0, _NB)
    def _edges(b):
        pltpu.sync_copy(val, acc_sh.at[rowv.at[b]], add=True)
    plsc.subcore_barrier()

    @pl.when(ci == 0)
    def _wb():
        pltpu.sync_copy(acc_sh.at[pl.ds(si * _RPT, _RPT)],
                        deg_hbm.at[pl.ds(si * _RPT, _RPT)])


def _deg_call(rowp):
    mesh = plsc.VectorSubcoreMesh(core_axis_name="c", subcore_axis_name="s",
                                  num_cores=_NC, num_subcores=_NS)
    f = pl.kernel(
        _deg_body,
        out_type=jax.ShapeDtypeStruct((_NACC, _F), jnp.float32),
        mesh=mesh,
        scratch_types=[
            pltpu.VMEM((_NB, _B), jnp.int32),
            pltpu.VMEM((_B, _F), jnp.float32),
            pltpu.VMEM_SHARED((_NACC, _F), jnp.float32),
            pltpu.SemaphoreType.DMA,
        ],
    )
    return f(rowp)


_GB = 8
_NG = _NB // _GB


def _spmm_body(n_chunks, zs_hbm, row_hbm, colp_hbm, out_hbm,
               rv0, rv1, cv0, cv1, gi0, gi1, gb0, gb1, zb, acc_sh,
               semi0, semi1, semg0, semg1, sems0, sems1):
    ci = lax.axis_index("c")
    si = lax.axis_index("s")

    @pl.loop(0, 32)
    def _zfill(r):
        for j in range(_F // 16):
            zb[r, pl.ds(16 * j, 16)] = jnp.zeros((16,), jnp.float32)

    def start_idx(g, rv, cv, semi):
        pltpu.async_copy(row_hbm.at[si, pl.ds(g * _GB, _GB)], rv, semi)
        pltpu.async_copy(colp_hbm.at[si, pl.ds(g * _GB, _GB)], cv, semi)

    def wait_idx(g, rv, cv, semi):
        pltpu.make_async_copy(row_hbm.at[si, pl.ds(g * _GB, _GB)], rv, semi).wait()
        pltpu.make_async_copy(colp_hbm.at[si, pl.ds(g * _GB, _GB)], cv, semi).wait()

    gis = (gi0, gi1)
    gbs = (gb0, gb1)
    semgs = (semg0, semg1)

    semss = (sems0, sems1)

    def process_group(rv, cv, base):
        for bb in range(_GB):
            a = bb % 2
            if bb >= 2:
                pltpu.make_async_copy(gbs[a], acc_sh.at[cv.at[bb - 2]],
                                      semss[a]).wait()
            for j in range(_B // 16):
                gis[a][pl.ds(16 * j, 16)] = rv[bb, pl.ds(16 * j, 16)] + base
            pltpu.async_copy(zs_hbm.at[gis[a]], gbs[a], semgs[a])
            if bb > 0:
                o = 1 - a
                pltpu.make_async_copy(zs_hbm.at[gis[o]], gbs[o], semgs[o]).wait()
                pltpu.async_copy(gbs[o], acc_sh.at[cv.at[bb - 1]], semss[o],
                                 add=True)
        last = (_GB - 1) % 2
        pltpu.make_async_copy(zs_hbm.at[gis[last]], gbs[last], semgs[last]).wait()
        pltpu.async_copy(gbs[last], acc_sh.at[cv.at[_GB - 1]], semss[last],
                         add=True)
        pltpu.make_async_copy(gbs[1 - last], acc_sh.at[cv.at[_GB - 2]],
                              semss[1 - last]).wait()
        pltpu.make_async_copy(gbs[last], acc_sh.at[cv.at[_GB - 1]],
                              semss[last]).wait()

    n_rounds = -(-n_chunks // _NC)
    for t in range(n_rounds):
        c = ci + _NC * t
        active = c < n_chunks

        @pl.when(active)
        def _zero_acc():
            @pl.loop(0, _RPT // 32)
            def _zero(z):
                pltpu.sync_copy(zb, acc_sh.at[pl.ds(si * _RPT + z * 32, 32)])
        plsc.subcore_barrier()

        @pl.when(active)
        def _edges():
            base = c * _N

            @pl.loop(0, _NG)
            def _u(g):
                start_idx(g, rv0, cv0, semi0)
                wait_idx(g, rv0, cv0, semi0)
                process_group(rv0, cv0, base)
        plsc.subcore_barrier()

        @pl.when(active)
        def _writeout():
            pltpu.sync_copy(acc_sh.at[pl.ds(si * _RPT, _RPT)],
                            out_hbm.at[c, pl.ds(si * _RPT, _RPT), :])
        plsc.subcore_barrier()


@functools.cache
def _spmm_fn(n_chunks):
    mesh = plsc.VectorSubcoreMesh(core_axis_name="c", subcore_axis_name="s",
                                  num_cores=_NC, num_subcores=_NS)
    return pl.kernel(
        functools.partial(_spmm_body, n_chunks),
        out_type=jax.ShapeDtypeStruct((n_chunks, _NACC, _F), jnp.float32),
        mesh=mesh,
        scratch_types=[
            pltpu.VMEM((_GB, _B), jnp.int32),
            pltpu.VMEM((_GB, _B), jnp.int32),
            pltpu.VMEM((_GB, _B), jnp.int32),
            pltpu.VMEM((_GB, _B), jnp.int32),
            pltpu.VMEM((_B,), jnp.int32),
            pltpu.VMEM((_B,), jnp.int32),
            pltpu.VMEM((_B, _F), jnp.float32),
            pltpu.VMEM((_B, _F), jnp.float32),
            pltpu.VMEM((32, _F), jnp.float32),
            pltpu.VMEM_SHARED((_NACC, _F), jnp.float32),
            pltpu.SemaphoreType.DMA,
            pltpu.SemaphoreType.DMA,
            pltpu.SemaphoreType.DMA,
            pltpu.SemaphoreType.DMA,
            pltpu.SemaphoreType.DMA,
            pltpu.SemaphoreType.DMA,
        ],
    )


def _spmm(zs_cm, row_r, colp_r):
    if False:
        row = row_r.reshape(-1)
        colp = colp_r.reshape(-1)
        nc = zs_cm.shape[0]
        acc = jnp.zeros((nc, _NACC, _F), jnp.float32)
        return acc.at[:, colp].add(zs_cm[:, row])
    nc = zs_cm.shape[0]
    return _spmm_fn(nc)(zs_cm.reshape(nc * _N, _F), row_r, colp_r)


_R = 1000


def _k0_body(deg_ref, x_ref, dis_ref, hs_ref):
    deg = deg_ref[:, :16]
    dis = jnp.where(deg > 0, jax.lax.rsqrt(jnp.where(deg > 0, deg, 1.0)), 0.0)
    dis_ref[...] = dis
    hs_ref[0] = x_ref[...] * dis[:, :1]


def _k0(deg, x):
    return pl.pallas_call(
        _k0_body,
        grid=(_N // _R,),
        in_specs=[pl.BlockSpec((_R, _F), lambda i: (i, 0)),
                  pl.BlockSpec((_R, 128), lambda i: (i, 0))],
        out_specs=[pl.BlockSpec((_R, 16), lambda i: (i, 0)),
                   pl.BlockSpec((1, _R, 128), lambda i: (0, i, 0))],
        out_shape=[jax.ShapeDtypeStruct((_N, 16), jnp.float32),
                   jax.ShapeDtypeStruct((1, _N, 128), jnp.float32)],
    )(deg, x)


def _kh_body(h_ref, dis_ref, hs_ref):
    hs_ref[0] = h_ref[...] * dis_ref[:, :1]


def _kh(h, dis):
    d = h.shape[1]
    nc = d // _F
    return pl.pallas_call(
        _kh_body,
        grid=(_N // _R, nc),
        in_specs=[pl.BlockSpec((_R, _F), lambda i, c: (i, c)),
                  pl.BlockSpec((_R, 16), lambda i, c: (i, 0))],
        out_specs=pl.BlockSpec((1, _R, _F), lambda i, c: (c, i, 0)),
        out_shape=jax.ShapeDtypeStruct((nc, _N, _F), jnp.float32),
    )(h, dis)


def _ka_body(h_ref, a1_ref, dis_ref, ps_ref):
    dis = dis_ref[:, :1]
    ps_ref[0] = dis * (h_ref[...] - dis * a1_ref[0])


def _ka(h, acc1, dis):
    d = h.shape[1]
    nc = d // _F
    return pl.pallas_call(
        _ka_body,
        grid=(_N // _R, nc),
        in_specs=[pl.BlockSpec((_R, _F), lambda i, c: (i, c)),
                  pl.BlockSpec((1, _R, _F), lambda i, c: (c, i, 0)),
                  pl.BlockSpec((_R, 16), lambda i, c: (i, 0))],
        out_specs=pl.BlockSpec((1, _R, _F), lambda i, c: (c, i, 0)),
        out_shape=jax.ShapeDtypeStruct((nc, _N, _F), jnp.float32),
    )(h, acc1, dis)


def _kb_body(nk, h_ref, a1_ref, a2_ref, dis_ref, m0_ref, m1_ref, m2_ref,
             bias_ref, out_ref):
    k = pl.program_id(1)
    dis = dis_ref[:, :1]
    h = h_ref[...]
    da1 = dis * a1_ref[0]
    p = h - da1
    t2 = h - 2.0 * da1 - 2.0 * (dis * a2_ref[0])
    dot = functools.partial(lax.dot_general,
                            dimension_numbers=(((1,), (0,)), ((), ())),
                            preferred_element_type=jnp.float32,
                            precision=lax.Precision.HIGHEST)
    contrib = dot(h, m0_ref[...]) + dot(p, m1_ref[...]) + dot(t2, m2_ref[...])

    @pl.when(k == 0)
    def _init():
        out_ref[...] = jnp.broadcast_to(bias_ref[...], out_ref.shape)

    out_ref[...] += contrib

    @pl.when(k == nk - 1)
    def _fin():
        out_ref[...] = jnp.maximum(out_ref[...], 0.0)


def _kb(h, acc1, acc2, dis, m0, m1, m2, bias):
    d = h.shape[1]
    D = m0.shape[1]
    nk = d // _F
    blk_h = pl.BlockSpec((_R, _F), lambda i, k: (i, k))
    blk_a = pl.BlockSpec((1, _R, _F), lambda i, k: (k, i, 0))
    blk_m = pl.BlockSpec((_F, D), lambda i, k: (k, 0))
    return pl.pallas_call(
        functools.partial(_kb_body, nk),
        grid=(_N // _R, nk),
        in_specs=[blk_h, blk_a, blk_a,
                  pl.BlockSpec((_R, 16), lambda i, k: (i, 0)),
                  blk_m, blk_m, blk_m,
                  pl.BlockSpec((1, D), lambda i, k: (0, 0))],
        out_specs=pl.BlockSpec((_R, D), lambda i, k: (i, 0)),
        out_shape=jax.ShapeDtypeStruct((_N, D), jnp.float32),
    )(h, acc1, acc2, dis, m0, m1, m2, bias)


@functools.cache
def _conv_mat_idx(li):
    ic, oc, kt, p, ilen = _LAYERS[li]
    olen = _OUT_LENS[li]
    rows, cols, wo, wi, wj = [], [], [], [], []
    for o in range(oc):
        for i in range(ic):
            for j in range(kt):
                for t in range(olen):
                    s = t + j - 2 * p
                    if 0 <= s < ilen:
                        rows.append(i * ilen + s)
                        cols.append(o * olen + t)
                        wo.append(o); wi.append(i); wj.append(j)
    return (np.asarray(rows), np.asarray(cols),
            np.asarray(wo), np.asarray(wi), np.asarray(wj))


def _conv_mats(li, W, cb, b):
    ic, oc, kt, p, ilen = _LAYERS[li]
    olen = _OUT_LENS[li]
    d_pad = _pad128(ic * ilen)
    D_pad = _pad128(oc * olen)
    rows, cols, wo, wi, wj = _conv_mat_idx(li)
    ms = []
    for k in range(3):
        m = jnp.zeros((d_pad, D_pad), jnp.float32)
        ms.append(m.at[rows, cols].set(W[k][wo, wi, wj]))
    bias = jnp.zeros((1, D_pad), jnp.float32)
    brow = (b.reshape(oc, olen) + cb.sum(axis=0)[:, None]).reshape(1, oc * olen)
    bias = bias.at[:, : oc * olen].set(brow)
    return ms, bias


def kernel(x, edge_index, W1, cb1, b1, W2, cb2, b2, W3, cb3, b3, W4, cb4, b4, W5, cb5, b5):
    row = edge_index[0]
    col = edge_index[1]
    selfe = row == col
    dummy = jnp.int32(_N)

    def prep(idx, pad):
        idx = jnp.concatenate([idx, jnp.full((_EPAD - _E,), pad, jnp.int32)])
        return idx.reshape(_NS, _NB, _B)

    row_g = prep(row, 0)
    rowp = prep(jnp.where(selfe, dummy, row), dummy)
    colp = prep(jnp.where(selfe, dummy, col), dummy)

    deg = _deg_call(rowp)
    dis, hs = _k0(deg, x)

    params = [(W1, cb1, b1), (W2, cb2, b2), (W3, cb3, b3), (W4, cb4, b4), (W5, cb5, b5)]
    h = x
    for li, (W, cb, b) in enumerate(params):
        ms, bias = _conv_mats(li, W, cb, b)
        acc1 = _spmm(hs, row_g, colp)
        ps = _ka(h, acc1, dis)
        acc2 = _spmm(ps, row_g, colp)
        h = _kb(h, acc1, acc2, dis, ms[0], ms[1], ms[2], bias)
        if li < 4:
            hs = _kh(h, dis)
    return h


# speedup vs baseline: 1.9985x; 1.9985x over previous
import functools
import numpy as np

import jax
import jax.numpy as jnp
from jax import lax
from jax.experimental import pallas as pl
from jax.experimental.pallas import tpu as pltpu
from jax.experimental.pallas import tpu_sc as plsc

_N = 10000
_NACC = 10240
_E = 320000
_NC, _NS = 2, 16
_NW = _NC * _NS
_B = 128
_NB = 160
_EPAD = _NS * _NB * _B
_RPT = _NACC // _NS
_F = 128

_LAYERS = [(4, 24, 5, 2, 32), (24, 32, 5, 1, 32), (32, 64, 5, 1, 30), (64, 64, 3, 0, 28), (64, 96, 3, 0, 26)]
_OUT_LENS = [32, 30, 28, 26, 24]


def _pad128(n):
    return (n + 127) // 128 * 128


def _deg_body(rowp_hbm, deg_hbm, rowv, val, acc_sh, sem):
    ci = lax.axis_index("c")
    si = lax.axis_index("s")

    cp = pltpu.async_copy(rowp_hbm.at[si], rowv, sem)

    @pl.loop(0, _B)
    def _fill0(r):
        for j in range(_F // 16):
            val[r, pl.ds(16 * j, 16)] = jnp.zeros((16,), jnp.float32)

    @pl.loop(0, _RPT // _B)
    def _zero(z):
        pltpu.sync_copy(val, acc_sh.at[pl.ds(si * _RPT + z * _B, _B)])

    @pl.loop(0, _B)
    def _fill1(r):
        for j in range(_F // 16):
            val[r, pl.ds(16 * j, 16)] = jnp.ones((16,), jnp.float32)
    cp.wait()
    plsc.subcore_barrier()

    @pl.loop(0, _NB)
    def _edges(b):
        pltpu.sync_copy(val, acc_sh.at[rowv.at[b]], add=True)
    plsc.subcore_barrier()

    @pl.when(ci == 0)
    def _wb():
        pltpu.sync_copy(acc_sh.at[pl.ds(si * _RPT, _RPT)],
                        deg_hbm.at[pl.ds(si * _RPT, _RPT)])


def _deg_call(rowp):
    mesh = plsc.VectorSubcoreMesh(core_axis_name="c", subcore_axis_name="s",
                                  num_cores=_NC, num_subcores=_NS)
    f = pl.kernel(
        _deg_body,
        out_type=jax.ShapeDtypeStruct((_NACC, _F), jnp.float32),
        mesh=mesh,
        scratch_types=[
            pltpu.VMEM((_NB, _B), jnp.int32),
            pltpu.VMEM((_B, _F), jnp.float32),
            pltpu.VMEM_SHARED((_NACC, _F), jnp.float32),
            pltpu.SemaphoreType.DMA,
        ],
    )
    return f(rowp)


_GB = 8
_NG = _NB // _GB


def _spmm_body(n_chunks, zs_hbm, row_hbm, colp_hbm, out_hbm,
               rv0, rv1, cv0, cv1, gi0, gi1, gb0, gb1, zb, acc_sh,
               semi0, semi1, semg0, semg1, sems0, sems1):
    ci = lax.axis_index("c")
    si = lax.axis_index("s")

    @pl.loop(0, 32)
    def _zfill(r):
        for j in range(_F // 16):
            zb[r, pl.ds(16 * j, 16)] = jnp.zeros((16,), jnp.float32)

    def start_idx(g, rv, cv, semi):
        pltpu.async_copy(row_hbm.at[si, pl.ds(g * _GB, _GB)], rv, semi)
        pltpu.async_copy(colp_hbm.at[si, pl.ds(g * _GB, _GB)], cv, semi)

    def wait_idx(g, rv, cv, semi):
        pltpu.make_async_copy(row_hbm.at[si, pl.ds(g * _GB, _GB)], rv, semi).wait()
        pltpu.make_async_copy(colp_hbm.at[si, pl.ds(g * _GB, _GB)], cv, semi).wait()

    gis = (gi0, gi1)
    gbs = (gb0, gb1)
    semgs = (semg0, semg1)

    semss = (sems0, sems1)

    def process_group(rv, cv, base):
        for bb in range(_GB):
            a = bb % 2
            if bb >= 2:
                pltpu.make_async_copy(gbs[a], acc_sh.at[cv.at[bb - 2]],
                                      semss[a]).wait()
            for j in range(_B // 16):
                gis[a][pl.ds(16 * j, 16)] = rv[bb, pl.ds(16 * j, 16)] + base
            pltpu.async_copy(zs_hbm.at[gis[a]], gbs[a], semgs[a])
            if bb > 0:
                o = 1 - a
                pltpu.make_async_copy(zs_hbm.at[gis[o]], gbs[o], semgs[o]).wait()
                pltpu.async_copy(gbs[o], acc_sh.at[cv.at[bb - 1]], semss[o],
                                 add=True)
        last = (_GB - 1) % 2
        pltpu.make_async_copy(zs_hbm.at[gis[last]], gbs[last], semgs[last]).wait()
        pltpu.async_copy(gbs[last], acc_sh.at[cv.at[_GB - 1]], semss[last],
                         add=True)
        pltpu.make_async_copy(gbs[1 - last], acc_sh.at[cv.at[_GB - 2]],
                              semss[1 - last]).wait()
        pltpu.make_async_copy(gbs[last], acc_sh.at[cv.at[_GB - 1]],
                              semss[last]).wait()

    n_rounds = -(-n_chunks // _NC)
    for t in range(n_rounds):
        c = ci + _NC * t
        active = c < n_chunks

        @pl.when(active)
        def _zero_acc():
            @pl.loop(0, _RPT // 32)
            def _zero(z):
                pltpu.sync_copy(zb, acc_sh.at[pl.ds(si * _RPT + z * 32, 32)])
        plsc.subcore_barrier()

        @pl.when(active)
        def _edges():
            base = c * _N

            @pl.loop(0, _NG)
            def _u(g):
                start_idx(g, rv0, cv0, semi0)
                wait_idx(g, rv0, cv0, semi0)
                process_group(rv0, cv0, base)
        plsc.subcore_barrier()

        @pl.when(active)
        def _writeout():
            pltpu.sync_copy(acc_sh.at[pl.ds(si * _RPT, _RPT)],
                            out_hbm.at[c, pl.ds(si * _RPT, _RPT), :])
        plsc.subcore_barrier()


@functools.cache
def _spmm_fn(n_chunks):
    mesh = plsc.VectorSubcoreMesh(core_axis_name="c", subcore_axis_name="s",
                                  num_cores=_NC, num_subcores=_NS)
    return pl.kernel(
        functools.partial(_spmm_body, n_chunks),
        out_type=jax.ShapeDtypeStruct((n_chunks, _NACC, _F), jnp.float32),
        mesh=mesh,
        scratch_types=[
            pltpu.VMEM((_GB, _B), jnp.int32),
            pltpu.VMEM((_GB, _B), jnp.int32),
            pltpu.VMEM((_GB, _B), jnp.int32),
            pltpu.VMEM((_GB, _B), jnp.int32),
            pltpu.VMEM((_B,), jnp.int32),
            pltpu.VMEM((_B,), jnp.int32),
            pltpu.VMEM((_B, _F), jnp.float32),
            pltpu.VMEM((_B, _F), jnp.float32),
            pltpu.VMEM((32, _F), jnp.float32),
            pltpu.VMEM_SHARED((_NACC, _F), jnp.float32),
            pltpu.SemaphoreType.DMA,
            pltpu.SemaphoreType.DMA,
            pltpu.SemaphoreType.DMA,
            pltpu.SemaphoreType.DMA,
            pltpu.SemaphoreType.DMA,
            pltpu.SemaphoreType.DMA,
        ],
    )


def _spmm(zs_cm, row_r, colp_r):
    if False:
        row = row_r.reshape(-1)
        colp = colp_r.reshape(-1)
        nc = zs_cm.shape[0]
        acc = jnp.zeros((nc, _NACC, _F), jnp.float32)
        return acc.at[:, colp].add(zs_cm[:, row])
    nc = zs_cm.shape[0]
    return _spmm_fn(nc)(zs_cm.reshape(nc * _N, _F), row_r, colp_r)


_R = 1000


def _k0_body(deg_ref, x_ref, dis_ref, hs_ref):
    deg = deg_ref[:, :16]
    dis = jnp.where(deg > 0, jax.lax.rsqrt(jnp.where(deg > 0, deg, 1.0)), 0.0)
    dis_ref[...] = dis
    hs_ref[0] = x_ref[...] * dis[:, :1]


def _k0(deg, x):
    return pl.pallas_call(
        _k0_body,
        grid=(_N // _R,),
        in_specs=[pl.BlockSpec((_R, _F), lambda i: (i, 0)),
                  pl.BlockSpec((_R, 128), lambda i: (i, 0))],
        out_specs=[pl.BlockSpec((_R, 16), lambda i: (i, 0)),
                   pl.BlockSpec((1, _R, 128), lambda i: (0, i, 0))],
        out_shape=[jax.ShapeDtypeStruct((_N, 16), jnp.float32),
                   jax.ShapeDtypeStruct((1, _N, 128), jnp.float32)],
    )(deg, x)


def _kh_body(h_ref, dis_ref, hs_ref):
    hs_ref[0] = h_ref[...] * dis_ref[:, :1]


def _kh(h, dis):
    d = h.shape[1]
    nc = d // _F
    return pl.pallas_call(
        _kh_body,
        grid=(_N // _R, nc),
        in_specs=[pl.BlockSpec((_R, _F), lambda i, c: (i, c)),
                  pl.BlockSpec((_R, 16), lambda i, c: (i, 0))],
        out_specs=pl.BlockSpec((1, _R, _F), lambda i, c: (c, i, 0)),
        out_shape=jax.ShapeDtypeStruct((nc, _N, _F), jnp.float32),
    )(h, dis)


def _ka_body(h_ref, a1_ref, dis_ref, ps_ref):
    dis = dis_ref[:, :1]
    ps_ref[0] = dis * (h_ref[...] - dis * a1_ref[0])


def _ka(h, acc1, dis):
    d = h.shape[1]
    nc = d // _F
    return pl.pallas_call(
        _ka_body,
        grid=(_N // _R, nc),
        in_specs=[pl.BlockSpec((_R, _F), lambda i, c: (i, c)),
                  pl.BlockSpec((1, _R, _F), lambda i, c: (c, i, 0)),
                  pl.BlockSpec((_R, 16), lambda i, c: (i, 0))],
        out_specs=pl.BlockSpec((1, _R, _F), lambda i, c: (c, i, 0)),
        out_shape=jax.ShapeDtypeStruct((nc, _N, _F), jnp.float32),
    )(h, acc1, dis)


def _kb_body(nk, h_ref, a1_ref, a2_ref, dis_ref, m0_ref, m1_ref, m2_ref,
             bias_ref, out_ref):
    k = pl.program_id(1)
    dis = dis_ref[:, :1]
    h = h_ref[...]
    da1 = dis * a1_ref[0]
    p = h - da1
    t2 = h - 2.0 * da1 - 2.0 * (dis * a2_ref[0])
    dot = functools.partial(lax.dot_general,
                            dimension_numbers=(((1,), (0,)), ((), ())),
                            preferred_element_type=jnp.float32,
                            precision=lax.Precision.HIGHEST)
    contrib = dot(h, m0_ref[...]) + dot(p, m1_ref[...]) + dot(t2, m2_ref[...])

    @pl.when(k == 0)
    def _init():
        out_ref[...] = jnp.broadcast_to(bias_ref[...], out_ref.shape)

    out_ref[...] += contrib

    @pl.when(k == nk - 1)
    def _fin():
        out_ref[...] = jnp.maximum(out_ref[...], 0.0)


def _kb(h, acc1, acc2, dis, m0, m1, m2, bias):
    d = h.shape[1]
    D = m0.shape[1]
    nk = d // _F
    blk_h = pl.BlockSpec((_R, _F), lambda i, k: (i, k))
    blk_a = pl.BlockSpec((1, _R, _F), lambda i, k: (k, i, 0))
    blk_m = pl.BlockSpec((_F, D), lambda i, k: (k, 0))
    return pl.pallas_call(
        functools.partial(_kb_body, nk),
        grid=(_N // _R, nk),
        in_specs=[blk_h, blk_a, blk_a,
                  pl.BlockSpec((_R, 16), lambda i, k: (i, 0)),
                  blk_m, blk_m, blk_m,
                  pl.BlockSpec((1, D), lambda i, k: (0, 0))],
        out_specs=pl.BlockSpec((_R, D), lambda i, k: (i, 0)),
        out_shape=jax.ShapeDtypeStruct((_N, D), jnp.float32),
    )(h, acc1, acc2, dis, m0, m1, m2, bias)


@functools.cache
def _conv_tap_tensor(li):
    ic, oc, kt, p, ilen = _LAYERS[li]
    olen = _OUT_LENS[li]
    E = np.zeros((kt, ilen, olen), np.float32)
    for j in range(kt):
        for t in range(olen):
            s = t + j - 2 * p
            if 0 <= s < ilen:
                E[j, s, t] = 1.0
    return E


def _conv_mats(li, W, cb, b):
    ic, oc, kt, p, ilen = _LAYERS[li]
    olen = _OUT_LENS[li]
    d_pad = _pad128(ic * ilen)
    D_pad = _pad128(oc * olen)
    E = jnp.asarray(_conv_tap_tensor(li))
    ms = []
    for k in range(3):
        m = jnp.einsum('oij,jst->isot', W[k], E,
                       preferred_element_type=jnp.float32)
        m = m.reshape(ic * ilen, oc * olen)
        m = jnp.pad(m, ((0, d_pad - ic * ilen), (0, D_pad - oc * olen)))
        ms.append(m)
    brow = (b.reshape(oc, olen) + cb.sum(axis=0)[:, None]).reshape(1, oc * olen)
    bias = jnp.pad(brow, ((0, 0), (0, D_pad - oc * olen)))
    return ms, bias


def kernel(x, edge_index, W1, cb1, b1, W2, cb2, b2, W3, cb3, b3, W4, cb4, b4, W5, cb5, b5):
    row = edge_index[0]
    col = edge_index[1]
    selfe = row == col
    dummy = jnp.int32(_N)

    def prep(idx, pad):
        idx = jnp.concatenate([idx, jnp.full((_EPAD - _E,), pad, jnp.int32)])
        return idx.reshape(_NS, _NB, _B)

    row_g = prep(row, 0)
    rowp = prep(jnp.where(selfe, dummy, row), dummy)
    colp = prep(jnp.where(selfe, dummy, col), dummy)

    deg = _deg_call(rowp)
    dis, hs = _k0(deg, x)

    params = [(W1, cb1, b1), (W2, cb2, b2), (W3, cb3, b3), (W4, cb4, b4), (W5, cb5, b5)]
    h = x
    for li, (W, cb, b) in enumerate(params):
        ms, bias = _conv_mats(li, W, cb, b)
        acc1 = _spmm(hs, row_g, colp)
        ps = _ka(h, acc1, dis)
        acc2 = _spmm(ps, row_g, colp)
        h = _kb(h, acc1, acc2, dis, ms[0], ms[1], ms[2], bias)
        if li < 4:
            hs = _kh(h, dis)
    return h


# speedup vs baseline: 2.5475x; 1.2747x over previous
import functools
import numpy as np

import jax
import jax.numpy as jnp
from jax import lax
from jax.experimental import pallas as pl
from jax.experimental.pallas import tpu as pltpu
from jax.experimental.pallas import tpu_sc as plsc

_N = 10000
_NACC = 10240
_E = 320000
_NC, _NS = 2, 16
_NW = _NC * _NS
_B = 128
_NB = 160
_EPAD = _NS * _NB * _B
_RPT = _NACC // _NS
_F = 128

_LAYERS = [(4, 24, 5, 2, 32), (24, 32, 5, 1, 32), (32, 64, 5, 1, 30), (64, 64, 3, 0, 28), (64, 96, 3, 0, 26)]
_OUT_LENS = [32, 30, 28, 26, 24]


def _pad128(n):
    return (n + 127) // 128 * 128


def _deg_body(rowp_hbm, deg_hbm, rowv, val, acc_sh, sem):
    ci = lax.axis_index("c")
    si = lax.axis_index("s")

    cp = pltpu.async_copy(rowp_hbm.at[si], rowv, sem)

    @pl.loop(0, _B)
    def _fill0(r):
        for j in range(_F // 16):
            val[r, pl.ds(16 * j, 16)] = jnp.zeros((16,), jnp.float32)

    @pl.loop(0, _RPT // _B)
    def _zero(z):
        pltpu.sync_copy(val, acc_sh.at[pl.ds(si * _RPT + z * _B, _B)])

    @pl.loop(0, _B)
    def _fill1(r):
        for j in range(_F // 16):
            val[r, pl.ds(16 * j, 16)] = jnp.ones((16,), jnp.float32)
    cp.wait()
    plsc.subcore_barrier()

    @pl.loop(0, _NB)
    def _edges(b):
        pltpu.sync_copy(val, acc_sh.at[rowv.at[b]], add=True)
    plsc.subcore_barrier()

    @pl.when(ci == 0)
    def _wb():
        pltpu.sync_copy(acc_sh.at[pl.ds(si * _RPT, _RPT)],
                        deg_hbm.at[pl.ds(si * _RPT, _RPT)])


def _deg_call(rowp):
    mesh = plsc.VectorSubcoreMesh(core_axis_name="c", subcore_axis_name="s",
                                  num_cores=_NC, num_subcores=_NS)
    f = pl.kernel(
        _deg_body,
        out_type=jax.ShapeDtypeStruct((_NACC, _F), jnp.float32),
        mesh=mesh,
        scratch_types=[
            pltpu.VMEM((_NB, _B), jnp.int32),
            pltpu.VMEM((_B, _F), jnp.float32),
            pltpu.VMEM_SHARED((_NACC, _F), jnp.float32),
            pltpu.SemaphoreType.DMA,
        ],
    )
    return f(rowp)


_GB = 8
_NG = _NB // _GB


_FH = _F // 2
_ZRT = _NACC // _NS


def _spmm_body(n_chunks, zs_hbm, row_hbm, colp_hbm, out_hbm,
               rv0, rv1, cv0, cv1, gb0, gb1, zb, zv, acc_sh,
               semi0, semi1, semg0, semg1, sems0, sems1, semz):
    ci = lax.axis_index("c")
    si = lax.axis_index("s")

    @pl.loop(0, 32)
    def _zfill(r):
        for j in range(_FH // 16):
            zb[r, pl.ds(16 * j, 16)] = jnp.zeros((16,), jnp.float32)

    def start_idx(g, rv, cv, semi):
        pltpu.async_copy(row_hbm.at[si, pl.ds(g * _GB, _GB)], rv, semi)
        pltpu.async_copy(colp_hbm.at[si, pl.ds(g * _GB, _GB)], cv, semi)

    def wait_idx(g, rv, cv, semi):
        pltpu.make_async_copy(row_hbm.at[si, pl.ds(g * _GB, _GB)], rv, semi).wait()
        pltpu.make_async_copy(colp_hbm.at[si, pl.ds(g * _GB, _GB)], cv, semi).wait()

    gbs = (gb0, gb1)
    semgs = (semg0, semg1)
    semss = (sems0, sems1)

    def process_group(rv, cv):
        for bb in range(_GB):
            a = bb % 2
            if bb >= 2:
                pltpu.make_async_copy(gbs[a], acc_sh.at[cv.at[bb - 2]],
                                      semss[a]).wait()
            pltpu.async_copy(zv.at[rv.at[bb]], gbs[a], semgs[a])
            if bb > 0:
                o = 1 - a
                pltpu.make_async_copy(zv.at[rv.at[bb - 1]], gbs[o], semgs[o]).wait()
                pltpu.async_copy(gbs[o], acc_sh.at[cv.at[bb - 1]], semss[o],
                                 add=True)
        last = (_GB - 1) % 2
        pltpu.make_async_copy(zv.at[rv.at[_GB - 1]], gbs[last], semgs[last]).wait()
        pltpu.async_copy(gbs[last], acc_sh.at[cv.at[_GB - 1]], semss[last],
                         add=True)
        pltpu.make_async_copy(gbs[1 - last], acc_sh.at[cv.at[_GB - 2]],
                              semss[1 - last]).wait()
        pltpu.make_async_copy(gbs[last], acc_sh.at[cv.at[_GB - 1]],
                              semss[last]).wait()

    for c in range(n_chunks):
        @pl.loop(0, _RPT // 32)
        def _zero(z):
            pltpu.sync_copy(zb, acc_sh.at[pl.ds(si * _RPT + z * 32, 32)])
        pltpu.async_copy(
            zs_hbm.at[pl.ds(c * _NACC + si * _ZRT, _ZRT), ci, :],
            zv.at[pl.ds(si * _ZRT, _ZRT)], semz)
        pltpu.make_async_copy(
            zs_hbm.at[pl.ds(c * _NACC + si * _ZRT, _ZRT), ci, :],
            zv.at[pl.ds(si * _ZRT, _ZRT)], semz).wait()
        plsc.subcore_barrier()

        @pl.loop(0, _NG)
        def _u(g):
            start_idx(g, rv0, cv0, semi0)
            wait_idx(g, rv0, cv0, semi0)
            process_group(rv0, cv0)
        plsc.subcore_barrier()

        pltpu.sync_copy(acc_sh.at[pl.ds(si * _RPT, _RPT)],
                        out_hbm.at[c, pl.ds(si * _RPT, _RPT), ci, :])
        plsc.subcore_barrier()


@functools.cache
def _spmm_fn(n_chunks):
    mesh = plsc.VectorSubcoreMesh(core_axis_name="c", subcore_axis_name="s",
                                  num_cores=_NC, num_subcores=_NS)
    return pl.kernel(
        functools.partial(_spmm_body, n_chunks),
        out_type=jax.ShapeDtypeStruct((n_chunks, _NACC, _NC, _FH), jnp.float32),
        mesh=mesh,
        compiler_params=pltpu.CompilerParams(use_tc_tiling_on_sc=False),
        scratch_types=[
            pltpu.VMEM((_GB, _B), jnp.int32),
            pltpu.VMEM((_GB, _B), jnp.int32),
            pltpu.VMEM((_GB, _B), jnp.int32),
            pltpu.VMEM((_GB, _B), jnp.int32),
            pltpu.VMEM((_B, _FH), jnp.float32),
            pltpu.VMEM((_B, _FH), jnp.float32),
            pltpu.VMEM((32, _FH), jnp.float32),
            pltpu.VMEM_SHARED((_NACC, _FH), jnp.float32),
            pltpu.VMEM_SHARED((_NACC, _FH), jnp.float32),
            pltpu.SemaphoreType.DMA,
            pltpu.SemaphoreType.DMA,
            pltpu.SemaphoreType.DMA,
            pltpu.SemaphoreType.DMA,
            pltpu.SemaphoreType.DMA,
            pltpu.SemaphoreType.DMA,
            pltpu.SemaphoreType.DMA,
        ],
    )


def _spmm(zs_cm, row_r, colp_r):
    if False:
        row = row_r.reshape(-1)
        colp = colp_r.reshape(-1)
        nc = zs_cm.shape[0]
        acc = jnp.zeros((nc, _NACC, _F), jnp.float32)
        return acc.at[:, colp].add(zs_cm[:, row])
    nc = zs_cm.shape[0]
    out = _spmm_fn(nc)(zs_cm.reshape(nc * _NACC, _NC, _FH), row_r, colp_r)
    return out.reshape(nc, _NACC, _F)


_R = 1000


def _k0_body(deg_ref, x_ref, dis_ref, hs_ref):
    deg = deg_ref[:, :16]
    dis = jnp.where(deg > 0, jax.lax.rsqrt(jnp.where(deg > 0, deg, 1.0)), 0.0)
    dis_ref[...] = dis
    hs_ref[0] = x_ref[...] * dis[:, :1]


def _k0(deg, x):
    return pl.pallas_call(
        _k0_body,
        grid=(_N // _R,),
        in_specs=[pl.BlockSpec((_R, _F), lambda i: (i, 0)),
                  pl.BlockSpec((_R, 128), lambda i: (i, 0))],
        out_specs=[pl.BlockSpec((_R, 16), lambda i: (i, 0)),
                   pl.BlockSpec((1, _R, 128), lambda i: (0, i, 0))],
        out_shape=[jax.ShapeDtypeStruct((_N, 16), jnp.float32),
                   jax.ShapeDtypeStruct((1, _NACC, 128), jnp.float32)],
    )(deg, x)


def _kh_body(h_ref, dis_ref, hs_ref):
    hs_ref[0] = h_ref[...] * dis_ref[:, :1]


def _kh(h, dis):
    d = h.shape[1]
    nc = d // _F
    return pl.pallas_call(
        _kh_body,
        grid=(_N // _R, nc),
        in_specs=[pl.BlockSpec((_R, _F), lambda i, c: (i, c)),
                  pl.BlockSpec((_R, 16), lambda i, c: (i, 0))],
        out_specs=pl.BlockSpec((1, _R, _F), lambda i, c: (c, i, 0)),
        out_shape=jax.ShapeDtypeStruct((nc, _NACC, _F), jnp.float32),
    )(h, dis)


def _ka_body(h_ref, a1_ref, dis_ref, ps_ref):
    dis = dis_ref[:, :1]
    ps_ref[0] = dis * (h_ref[...] - dis * a1_ref[0])


def _ka(h, acc1, dis):
    d = h.shape[1]
    nc = d // _F
    return pl.pallas_call(
        _ka_body,
        grid=(_N // _R, nc),
        in_specs=[pl.BlockSpec((_R, _F), lambda i, c: (i, c)),
                  pl.BlockSpec((1, _R, _F), lambda i, c: (c, i, 0)),
                  pl.BlockSpec((_R, 16), lambda i, c: (i, 0))],
        out_specs=pl.BlockSpec((1, _R, _F), lambda i, c: (c, i, 0)),
        out_shape=jax.ShapeDtypeStruct((nc, _NACC, _F), jnp.float32),
    )(h, acc1, dis)


def _kb_body(nk, h_ref, a1_ref, a2_ref, dis_ref, m0_ref, m1_ref, m2_ref,
             bias_ref, out_ref):
    k = pl.program_id(1)
    dis = dis_ref[:, :1]
    h = h_ref[...]
    da1 = dis * a1_ref[0]
    p = h - da1
    t2 = h - 2.0 * da1 - 2.0 * (dis * a2_ref[0])
    dot = functools.partial(lax.dot_general,
                            dimension_numbers=(((1,), (0,)), ((), ())),
                            preferred_element_type=jnp.float32,
                            precision=lax.Precision.HIGHEST)
    contrib = dot(h, m0_ref[...]) + dot(p, m1_ref[...]) + dot(t2, m2_ref[...])

    @pl.when(k == 0)
    def _init():
        out_ref[...] = jnp.broadcast_to(bias_ref[...], out_ref.shape)

    out_ref[...] += contrib

    @pl.when(k == nk - 1)
    def _fin():
        out_ref[...] = jnp.maximum(out_ref[...], 0.0)


def _kb(h, acc1, acc2, dis, m0, m1, m2, bias):
    d = h.shape[1]
    D = m0.shape[1]
    nk = d // _F
    blk_h = pl.BlockSpec((_R, _F), lambda i, k: (i, k))
    blk_a = pl.BlockSpec((1, _R, _F), lambda i, k: (k, i, 0))
    blk_m = pl.BlockSpec((_F, D), lambda i, k: (k, 0))
    return pl.pallas_call(
        functools.partial(_kb_body, nk),
        grid=(_N // _R, nk),
        in_specs=[blk_h, blk_a, blk_a,
                  pl.BlockSpec((_R, 16), lambda i, k: (i, 0)),
                  blk_m, blk_m, blk_m,
                  pl.BlockSpec((1, D), lambda i, k: (0, 0))],
        out_specs=pl.BlockSpec((_R, D), lambda i, k: (i, 0)),
        out_shape=jax.ShapeDtypeStruct((_N, D), jnp.float32),
    )(h, acc1, acc2, dis, m0, m1, m2, bias)


@functools.cache
def _conv_tap_tensor(li):
    ic, oc, kt, p, ilen = _LAYERS[li]
    olen = _OUT_LENS[li]
    E = np.zeros((kt, ilen, olen), np.float32)
    for j in range(kt):
        for t in range(olen):
            s = t + j - 2 * p
            if 0 <= s < ilen:
                E[j, s, t] = 1.0
    return E


def _conv_mats(li, W, cb, b):
    ic, oc, kt, p, ilen = _LAYERS[li]
    olen = _OUT_LENS[li]
    d_pad = _pad128(ic * ilen)
    D_pad = _pad128(oc * olen)
    E = jnp.asarray(_conv_tap_tensor(li))
    ms = []
    for k in range(3):
        m = jnp.einsum('oij,jst->isot', W[k], E,
                       preferred_element_type=jnp.float32)
        m = m.reshape(ic * ilen, oc * olen)
        m = jnp.pad(m, ((0, d_pad - ic * ilen), (0, D_pad - oc * olen)))
        ms.append(m)
    brow = (b.reshape(oc, olen) + cb.sum(axis=0)[:, None]).reshape(1, oc * olen)
    bias = jnp.pad(brow, ((0, 0), (0, D_pad - oc * olen)))
    return ms, bias


def kernel(x, edge_index, W1, cb1, b1, W2, cb2, b2, W3, cb3, b3, W4, cb4, b4, W5, cb5, b5):
    row = edge_index[0]
    col = edge_index[1]
    selfe = row == col
    dummy = jnp.int32(_N)

    def prep(idx, pad):
        idx = jnp.concatenate([idx, jnp.full((_EPAD - _E,), pad, jnp.int32)])
        return idx.reshape(_NS, _NB, _B)

    row_g = prep(row, 0)
    rowp = prep(jnp.where(selfe, dummy, row), dummy)
    colp = prep(jnp.where(selfe, dummy, col), dummy)

    deg = _deg_call(rowp)
    dis, hs = _k0(deg, x)

    params = [(W1, cb1, b1), (W2, cb2, b2), (W3, cb3, b3), (W4, cb4, b4), (W5, cb5, b5)]
    h = x
    for li, (W, cb, b) in enumerate(params):
        ms, bias = _conv_mats(li, W, cb, b)
        acc1 = _spmm(hs, row_g, colp)
        ps = _ka(h, acc1, dis)
        acc2 = _spmm(ps, row_g, colp)
        h = _kb(h, acc1, acc2, dis, ms[0], ms[1], ms[2], bias)
        if li < 4:
            hs = _kh(h, dis)
    return h
